# baseline (speedup 1.0000x reference)
import functools

import jax
import jax.numpy as jnp
from jax import lax
from jax.experimental import pallas as pl
from jax.experimental.pallas import tpu as pltpu
from jax.experimental.pallas import tpu_sc as plsc

_N, _E, _D, _G, _C = 10000, 320000, 128, 64, 10
_NC, _NS = 2, 16
_K = 80
_NCH = 128
_NSL = 8
_CPS = _NCH // _NSL
_R = 4
_EPT = _K * _NCH
_TOT = _NC * _NS * _EPT
_NP = 10112
_RPT = _NP // _NS
_NDEG = 10240
_DPT = _NDEG // _NS

def _deg_body(dst_hbm, zeros_hbm, out_hbm, acc_sh, dst_v, ones_v):
    c = lax.axis_index("c")
    s = lax.axis_index("s")
    off = s * _DPT
    pltpu.sync_copy(zeros_hbm.at[pl.ds(off, _DPT)], acc_sh.at[pl.ds(off, _DPT)])
    pltpu.sync_copy(dst_hbm.at[c, s], dst_v)
    for i in range(_K // 16):
        ones_v[pl.ds(i * 16, 16)] = jnp.ones((16,), jnp.float32)
    plsc.subcore_barrier()

    def body(j, carry):
        pltpu.sync_copy(ones_v, acc_sh.at[dst_v.at[j]], add=True)
        return carry

    lax.fori_loop(0, _NCH, body, 0)
    plsc.subcore_barrier()
    pltpu.sync_copy(acc_sh.at[pl.ds(off, _DPT)], out_hbm.at[c, pl.ds(off, _DPT)])


@functools.lru_cache(maxsize=None)
def _get_deg_call():
    mesh = plsc.VectorSubcoreMesh(core_axis_name="c", subcore_axis_name="s")
    return pl.kernel(
        _deg_body,
        out_type=jax.ShapeDtypeStruct((_NC, _NDEG), jnp.float32),
        mesh=mesh,
        scratch_types=[
            pltpu.VMEM_SHARED((_NDEG,), jnp.float32),
            pltpu.VMEM((_NCH, _K), jnp.int32),
            pltpu.VMEM((_K,), jnp.float32),
        ],
    )


def _scat_body(hw_hbm, src_hbm, dst_hbm, zeros_hbm, out_hbm,
               acc_sh, src_v, dst_v, rows_v,
               sg0, sg1, sg2, sg3, ss0, ss1, ss2, ss3, si0, si1):
    c = lax.axis_index("c")
    s = lax.axis_index("s")
    off = s * _RPT
    sg = (sg0, sg1, sg2, sg3)
    ss = (ss0, ss1, ss2, ss3)
    si = (si0, si1)
    init = pltpu.async_copy(
        zeros_hbm.at[pl.ds(off, _RPT)], acc_sh.at[pl.ds(off, _RPT)], sg[2])

    def idx_fetch(q, p):
        qb = q * _CPS
        pltpu.async_copy(src_hbm.at[c, s, pl.ds(qb, _CPS)], src_v.at[p], si[p])
        pltpu.async_copy(dst_hbm.at[c, s, pl.ds(qb, _CPS)], dst_v.at[p], si[p])

    def idx_wait(q, p):
        qb = q * _CPS
        pltpu.make_async_copy(
            src_hbm.at[c, s, pl.ds(qb, _CPS)], src_v.at[p], si[p]).wait()
        pltpu.make_async_copy(
            dst_hbm.at[c, s, pl.ds(qb, _CPS)], dst_v.at[p], si[p]).wait()

    idx_fetch(0, 0)
    for q in range(_NSL):
        p = q % 2
        if q + 1 < _NSL:
            idx_fetch(q + 1, 1 - p)
        idx_wait(q, p)
        if q == 0:
            init.wait()
            plsc.subcore_barrier()
        pltpu.async_copy(hw_hbm.at[src_v.at[p, 0]], rows_v.at[0], sg[0])
        pltpu.async_copy(hw_hbm.at[src_v.at[p, 1]], rows_v.at[1], sg[1])

        def body(t, carry):
            j = t * _R
            for b in range(_R):
                jj = j + b
                bp = (b + 2) % _R
                pltpu.make_async_copy(
                    hw_hbm.at[src_v.at[p, jj]], rows_v.at[b], sg[b]).wait()
                pltpu.async_copy(rows_v.at[b], acc_sh.at[dst_v.at[p, jj]],
                                 ss[b], add=True)

                @pl.when(jj >= 2)
                def _():
                    pltpu.make_async_copy(
                        rows_v.at[bp], acc_sh.at[dst_v.at[p, jj - 2]],
                        ss[bp]).wait()

                @pl.when(jj + 2 < _CPS)
                def _():
                    pltpu.async_copy(
                        hw_hbm.at[src_v.at[p, jj + 2]], rows_v.at[bp], sg[bp])
            return carry

        lax.fori_loop(0, _CPS // _R, body, 0)
        pltpu.make_async_copy(
            rows_v.at[(_CPS - 2) % _R], acc_sh.at[dst_v.at[p, _CPS - 2]],
            ss[(_CPS - 2) % _R]).wait()
        pltpu.make_async_copy(
            rows_v.at[(_CPS - 1) % _R], acc_sh.at[dst_v.at[p, _CPS - 1]],
            ss[(_CPS - 1) % _R]).wait()
    plsc.subcore_barrier()
    pltpu.sync_copy(acc_sh.at[pl.ds(off, _RPT)], out_hbm.at[c, pl.ds(off, _RPT)])


@functools.lru_cache(maxsize=None)
def _get_scat_call():
    mesh = plsc.VectorSubcoreMesh(core_axis_name="c", subcore_axis_name="s")
    return pl.kernel(
        _scat_body,
        out_type=jax.ShapeDtypeStruct((_NC, _NP, _D), jnp.float32),
        mesh=mesh,
        scratch_types=[
            pltpu.VMEM_SHARED((_NP, _D), jnp.float32),
            pltpu.VMEM((2, _CPS, _K), jnp.int32),
            pltpu.VMEM((2, _CPS, _K), jnp.int32),
            pltpu.VMEM((_R, _K, _D), jnp.float32),
        ] + [pltpu.SemaphoreType.DMA] * (2 * _R + 2),
    )


def _row_mask():
    return lax.broadcasted_iota(jnp.int32, (_NP, 1), 0) < _N


def _tc1_body(degp_ref, xp_ref, embW_ref, embb_ref, W0_ref, dinv_ref, hw0_ref):
    deg = degp_ref[0] + degp_ref[1] + 1.0
    dinv = lax.rsqrt(jnp.maximum(deg, 1.0))
    dinv = jnp.where(_row_mask(), dinv, 0.0)
    dinv_ref[...] = dinv
    h0 = jnp.dot(xp_ref[...], embW_ref[...],
                 preferred_element_type=jnp.float32) + embb_ref[...]
    hw0 = jnp.dot(h0, W0_ref[...], preferred_element_type=jnp.float32)
    hw0_ref[...] = dinv * hw0


_tc1_call = pl.pallas_call(
    _tc1_body,
    out_shape=[
        jax.ShapeDtypeStruct((_NP, 1), jnp.float32),
        jax.ShapeDtypeStruct((_NP, _D), jnp.float32),
    ],
)


def _tc2_body(accp_ref, hwp_ref, dinv_ref, b_ref, g_ref, be_ref, Wn_ref,
              hwn_ref):
    dinv = dinv_ref[...]
    mask = _row_mask()
    out = dinv * (accp_ref[0] + accp_ref[1] + hwp_ref[...]) + b_ref[...]
    out = jnp.where(mask, out, 0.0)
    m = jnp.sum(out, axis=0, keepdims=True) * (1.0 / _N)
    cen = out - m
    cenm = jnp.where(mask, cen, 0.0)
    v = jnp.sum(cenm * cenm, axis=0, keepdims=True) * (1.0 / _N)
    h = g_ref[...] * cen / jnp.sqrt(v + 1e-5) + be_ref[...]
    h = jnp.where(mask, jnp.maximum(h, 0.0), 0.0)
    hwn_ref[...] = dinv * jnp.dot(h, Wn_ref[...],
                                  preferred_element_type=jnp.float32)


_tc2_call = pl.pallas_call(
    _tc2_body,
    out_shape=jax.ShapeDtypeStruct((_NP, _D), jnp.float32),
)


def _tc3_body(accp_ref, hwp_ref, dinv_ref, b_ref, batch_ref,
              cW1_ref, cb1_ref, cW2_ref, cb2_ref, cW3_ref, cb3_ref, out_ref):
    h = dinv_ref[...] * (accp_ref[0] + accp_ref[1] + hwp_ref[...]) + b_ref[...]
    gids = lax.broadcasted_iota(jnp.int32, (_G, _NP), 0)
    M = (batch_ref[...] == gids).astype(jnp.float32)
    sums = jnp.dot(M, h, preferred_element_type=jnp.float32)
    cnt = jnp.sum(M, axis=1, keepdims=True)
    hg = sums / jnp.maximum(cnt, 1.0)
    z = jnp.maximum(jnp.dot(hg, cW1_ref[...],
                            preferred_element_type=jnp.float32)
                    + cb1_ref[...], 0.0)
    z = jnp.maximum(jnp.dot(z, cW2_ref[...],
                            preferred_element_type=jnp.float32)
                    + cb2_ref[...], 0.0)
    out_ref[...] = jnp.dot(z, cW3_ref[...],
                           preferred_element_type=jnp.float32) + cb3_ref[...]


_tc3_call = pl.pallas_call(
    _tc3_body,
    out_shape=jax.ShapeDtypeStruct((_G, _C), jnp.float32),
)


def kernel(x, edge_index, batch, emb_W, emb_b, W0, b0, g0, be0,
           W1, b1, g1, be1, W2, b2, cW1, cb1, cW2, cb2, cW3, cb3):
    f32 = jnp.float32
    pad = _TOT - _E
    pad_idx = _N + (jnp.arange(pad, dtype=jnp.int32) % 16)
    src_p = jnp.concatenate([edge_index[0], pad_idx]).reshape(_NC, _NS, _NCH, _K)
    dst_p = jnp.concatenate([edge_index[1], pad_idx]).reshape(_NC, _NS, _NCH, _K)
    zeros1 = jnp.zeros((_NDEG,), f32)
    zeros2 = jnp.zeros((_NP, _D), f32)
    xp = jnp.pad(x.astype(f32), ((0, _NP - _N), (0, 0)))
    batchp = jnp.pad(batch, (0, _NP - _N), constant_values=_G).reshape(1, _NP)

    degp = _get_deg_call()(dst_p, zeros1)
    degp3 = degp[:, :_NP, None]

    dinv, hw0 = _tc1_call(degp3, xp, emb_W, emb_b.reshape(1, _D), W0)

    scat = _get_scat_call()
    acc0 = scat(hw0, src_p, dst_p, zeros2)
    hw1 = _tc2_call(acc0, hw0, dinv, b0.reshape(1, _D), g0.reshape(1, _D),
                    be0.reshape(1, _D), W1)
    acc1 = scat(hw1, src_p, dst_p, zeros2)
    hw2 = _tc2_call(acc1, hw1, dinv, b1.reshape(1, _D), g1.reshape(1, _D),
                    be1.reshape(1, _D), W2)
    acc2 = scat(hw2, src_p, dst_p, zeros2)

    return _tc3_call(acc2, hw2, dinv, b2.reshape(1, _D), batchp,
                     cW1, cb1.reshape(1, _D // 2), cW2,
                     cb2.reshape(1, _D // 4), cW3, cb3.reshape(1, _C))


# speedup vs baseline: 1.0145x; 1.0145x over previous
import functools

import jax
import jax.numpy as jnp
from jax import lax
from jax.experimental import pallas as pl
from jax.experimental.pallas import tpu as pltpu
from jax.experimental.pallas import tpu_sc as plsc

_N, _E, _D, _G, _C = 10000, 320000, 128, 64, 10
_NC, _NS = 2, 16
_K = 80
_NCH = 128
_NSL = 8
_CPS = _NCH // _NSL
_R = 4
_EPT = _K * _NCH
_TOT = _NC * _NS * _EPT
_NP = 10112
_RPT = _NP // _NS
_NDEG = 10240
_DPT = _NDEG // _NS

def _deg_body(dst_hbm, zeros_hbm, out_hbm, acc_sh, dst_v, ones_v, sem):
    c = lax.axis_index("c")
    s = lax.axis_index("s")
    off = s * _DPT
    pltpu.sync_copy(zeros_hbm.at[pl.ds(off, _DPT)], acc_sh.at[pl.ds(off, _DPT)])
    pltpu.sync_copy(dst_hbm.at[c, s], dst_v)
    for i in range(_K // 16):
        ones_v[pl.ds(i * 16, 16)] = jnp.ones((16,), jnp.float32)
    plsc.subcore_barrier()

    def body(j, carry):
        @pl.when(j >= 8)
        def _():
            pltpu.make_async_copy(
                ones_v, acc_sh.at[dst_v.at[j - 8]], sem).wait()
        pltpu.async_copy(ones_v, acc_sh.at[dst_v.at[j]], sem, add=True)
        return carry

    lax.fori_loop(0, _NCH, body, 0)
    for j in range(_NCH - 8, _NCH):
        pltpu.make_async_copy(ones_v, acc_sh.at[dst_v.at[j]], sem).wait()
    plsc.subcore_barrier()
    pltpu.sync_copy(acc_sh.at[pl.ds(off, _DPT)], out_hbm.at[c, pl.ds(off, _DPT)])


@functools.lru_cache(maxsize=None)
def _get_deg_call():
    mesh = plsc.VectorSubcoreMesh(core_axis_name="c", subcore_axis_name="s")
    return pl.kernel(
        _deg_body,
        out_type=jax.ShapeDtypeStruct((_NC, _NDEG), jnp.float32),
        mesh=mesh,
        scratch_types=[
            pltpu.VMEM_SHARED((_NDEG,), jnp.float32),
            pltpu.VMEM((_NCH, _K), jnp.int32),
            pltpu.VMEM((_K,), jnp.float32),
            pltpu.SemaphoreType.DMA,
        ],
    )


def _scat_body(hw_hbm, src_hbm, dst_hbm, zeros_hbm, out_hbm,
               acc_sh, src_v, dst_v, rows_v,
               sg0, sg1, sg2, sg3, ss0, ss1, ss2, ss3, si0, si1):
    c = lax.axis_index("c")
    s = lax.axis_index("s")
    off = s * _RPT
    sg = (sg0, sg1, sg2, sg3)
    ss = (ss0, ss1, ss2, ss3)
    si = (si0, si1)
    init = pltpu.async_copy(
        zeros_hbm.at[pl.ds(off, _RPT)], acc_sh.at[pl.ds(off, _RPT)], sg[2])

    def idx_fetch(q, p):
        qb = q * _CPS
        pltpu.async_copy(src_hbm.at[c, s, pl.ds(qb, _CPS)], src_v.at[p], si[p])
        pltpu.async_copy(dst_hbm.at[c, s, pl.ds(qb, _CPS)], dst_v.at[p], si[p])

    def idx_wait(q, p):
        qb = q * _CPS
        pltpu.make_async_copy(
            src_hbm.at[c, s, pl.ds(qb, _CPS)], src_v.at[p], si[p]).wait()
        pltpu.make_async_copy(
            dst_hbm.at[c, s, pl.ds(qb, _CPS)], dst_v.at[p], si[p]).wait()

    idx_fetch(0, 0)
    for q in range(_NSL):
        p = q % 2
        if q + 1 < _NSL:
            idx_fetch(q + 1, 1 - p)
        idx_wait(q, p)
        if q == 0:
            init.wait()
            plsc.subcore_barrier()
        pltpu.async_copy(hw_hbm.at[src_v.at[p, 0]], rows_v.at[0], sg[0])
        pltpu.async_copy(hw_hbm.at[src_v.at[p, 1]], rows_v.at[1], sg[1])

        def body(t, carry):
            j = t * _R
            for b in range(_R):
                jj = j + b
                bp = (b + 2) % _R
                pltpu.make_async_copy(
                    hw_hbm.at[src_v.at[p, jj]], rows_v.at[b], sg[b]).wait()
                pltpu.async_copy(rows_v.at[b], acc_sh.at[dst_v.at[p, jj]],
                                 ss[b], add=True)

                @pl.when(jj >= 2)
                def _():
                    pltpu.make_async_copy(
                        rows_v.at[bp], acc_sh.at[dst_v.at[p, jj - 2]],
                        ss[bp]).wait()

                @pl.when(jj + 2 < _CPS)
                def _():
                    pltpu.async_copy(
                        hw_hbm.at[src_v.at[p, jj + 2]], rows_v.at[bp], sg[bp])
            return carry

        lax.fori_loop(0, _CPS // _R, body, 0)
        pltpu.make_async_copy(
            rows_v.at[(_CPS - 2) % _R], acc_sh.at[dst_v.at[p, _CPS - 2]],
            ss[(_CPS - 2) % _R]).wait()
        pltpu.make_async_copy(
            rows_v.at[(_CPS - 1) % _R], acc_sh.at[dst_v.at[p, _CPS - 1]],
            ss[(_CPS - 1) % _R]).wait()
    plsc.subcore_barrier()
    pltpu.sync_copy(acc_sh.at[pl.ds(off, _RPT)], out_hbm.at[c, pl.ds(off, _RPT)])


@functools.lru_cache(maxsize=None)
def _get_scat_call():
    mesh = plsc.VectorSubcoreMesh(core_axis_name="c", subcore_axis_name="s")
    return pl.kernel(
        _scat_body,
        out_type=jax.ShapeDtypeStruct((_NC, _NP, _D), jnp.float32),
        mesh=mesh,
        scratch_types=[
            pltpu.VMEM_SHARED((_NP, _D), jnp.float32),
            pltpu.VMEM((2, _CPS, _K), jnp.int32),
            pltpu.VMEM((2, _CPS, _K), jnp.int32),
            pltpu.VMEM((_R, _K, _D), jnp.float32),
        ] + [pltpu.SemaphoreType.DMA] * (2 * _R + 2),
    )


def _row_mask():
    return lax.broadcasted_iota(jnp.int32, (_NP, 1), 0) < _N


def _tc1_body(degp_ref, xp_ref, embW_ref, embb_ref, W0_ref, dinv_ref, hw0_ref):
    deg = degp_ref[0] + degp_ref[1] + 1.0
    dinv = lax.rsqrt(jnp.maximum(deg, 1.0))
    dinv = jnp.where(_row_mask(), dinv, 0.0)
    dinv_ref[...] = dinv
    h0 = jnp.dot(xp_ref[...], embW_ref[...],
                 preferred_element_type=jnp.float32) + embb_ref[...]
    hw0 = jnp.dot(h0, W0_ref[...], preferred_element_type=jnp.float32)
    hw0_ref[...] = dinv * hw0


_tc1_call = pl.pallas_call(
    _tc1_body,
    out_shape=[
        jax.ShapeDtypeStruct((_NP, 1), jnp.float32),
        jax.ShapeDtypeStruct((_NP, _D), jnp.float32),
    ],
)


def _tc2_body(accp_ref, hwp_ref, dinv_ref, b_ref, g_ref, be_ref, Wn_ref,
              hwn_ref):
    dinv = dinv_ref[...]
    mask = _row_mask()
    out = dinv * (accp_ref[0] + accp_ref[1] + hwp_ref[...]) + b_ref[...]
    out = jnp.where(mask, out, 0.0)
    m = jnp.sum(out, axis=0, keepdims=True) * (1.0 / _N)
    cen = out - m
    cenm = jnp.where(mask, cen, 0.0)
    v = jnp.sum(cenm * cenm, axis=0, keepdims=True) * (1.0 / _N)
    h = g_ref[...] * cen / jnp.sqrt(v + 1e-5) + be_ref[...]
    h = jnp.where(mask, jnp.maximum(h, 0.0), 0.0)
    hwn_ref[...] = dinv * jnp.dot(h, Wn_ref[...],
                                  preferred_element_type=jnp.float32)


_tc2_call = pl.pallas_call(
    _tc2_body,
    out_shape=jax.ShapeDtypeStruct((_NP, _D), jnp.float32),
)


def _tc3_body(accp_ref, hwp_ref, dinv_ref, b_ref, batch_ref,
              cW1_ref, cb1_ref, cW2_ref, cb2_ref, cW3_ref, cb3_ref, out_ref):
    h = dinv_ref[...] * (accp_ref[0] + accp_ref[1] + hwp_ref[...]) + b_ref[...]
    gids = lax.broadcasted_iota(jnp.int32, (_G, _NP), 0)
    M = (batch_ref[...] == gids).astype(jnp.float32)
    sums = jnp.dot(M, h, preferred_element_type=jnp.float32)
    cnt = jnp.sum(M, axis=1, keepdims=True)
    hg = sums / jnp.maximum(cnt, 1.0)
    z = jnp.maximum(jnp.dot(hg, cW1_ref[...],
                            preferred_element_type=jnp.float32)
                    + cb1_ref[...], 0.0)
    z = jnp.maximum(jnp.dot(z, cW2_ref[...],
                            preferred_element_type=jnp.float32)
                    + cb2_ref[...], 0.0)
    out_ref[...] = jnp.dot(z, cW3_ref[...],
                           preferred_element_type=jnp.float32) + cb3_ref[...]


_tc3_call = pl.pallas_call(
    _tc3_body,
    out_shape=jax.ShapeDtypeStruct((_G, _C), jnp.float32),
)


def kernel(x, edge_index, batch, emb_W, emb_b, W0, b0, g0, be0,
           W1, b1, g1, be1, W2, b2, cW1, cb1, cW2, cb2, cW3, cb3):
    f32 = jnp.float32
    pad = _TOT - _E
    pad_idx = _N + (jnp.arange(pad, dtype=jnp.int32) % 16)
    src_p = jnp.concatenate([edge_index[0], pad_idx]).reshape(_NC, _NS, _NCH, _K)
    dst_p = jnp.concatenate([edge_index[1], pad_idx]).reshape(_NC, _NS, _NCH, _K)
    zeros1 = jnp.zeros((_NDEG,), f32)
    zeros2 = jnp.zeros((_NP, _D), f32)
    xp = jnp.pad(x.astype(f32), ((0, _NP - _N), (0, 0)))
    batchp = jnp.pad(batch, (0, _NP - _N), constant_values=_G).reshape(1, _NP)

    degp = _get_deg_call()(dst_p, zeros1)
    degp3 = degp[:, :_NP, None]

    dinv, hw0 = _tc1_call(degp3, xp, emb_W, emb_b.reshape(1, _D), W0)

    scat = _get_scat_call()
    acc0 = scat(hw0, src_p, dst_p, zeros2)
    hw1 = _tc2_call(acc0, hw0, dinv, b0.reshape(1, _D), g0.reshape(1, _D),
                    be0.reshape(1, _D), W1)
    acc1 = scat(hw1, src_p, dst_p, zeros2)
    hw2 = _tc2_call(acc1, hw1, dinv, b1.reshape(1, _D), g1.reshape(1, _D),
                    be1.reshape(1, _D), W2)
    acc2 = scat(hw2, src_p, dst_p, zeros2)

    return _tc3_call(acc2, hw2, dinv, b2.reshape(1, _D), batchp,
                     cW1, cb1.reshape(1, _D // 2), cW2,
                     cb2.reshape(1, _D // 4), cW3, cb3.reshape(1, _C))
